# baseline (speedup 1.0000x reference)
import functools

import jax
import jax.numpy as jnp
import numpy as np
from jax import lax
from jax.experimental import pallas as pl
from jax.experimental.pallas import tpu as pltpu
from jax.experimental.pallas import tpu_sc as plsc

N = 10000
E = 320000
D = 128
NW = 32
LB = 128
BPW = 81
SLOP = 3
NE_P = (NW * BPW + SLOP) * LB
NP = 10112
RPT = NP // 16

_PAD_SRC = np.asarray((np.arange(NE_P - E) * 997) % N, dtype=np.int32)
_PAD_DST = np.asarray(N + np.arange(NE_P - E) % (NP - N), dtype=np.int32)


def _segment_sum_sc(x, edge_index, pad_src, pad_dst):
    mesh = plsc.VectorSubcoreMesh(core_axis_name="c", subcore_axis_name="s")

    @functools.partial(
        pl.kernel,
        mesh=mesh,
        out_type=jax.ShapeDtypeStruct((2 * NP, D), jnp.float32),
        scratch_types=[
            pltpu.VMEM((3, 1, 2, LB), jnp.int32),
            pltpu.VMEM((3, LB, D), jnp.float32),
            pltpu.VMEM_SHARED((NP, D), jnp.float32),
            pltpu.SemaphoreType.DMA,
            pltpu.SemaphoreType.DMA,
        ],
    )
    def seg_kernel(x_hbm, edge_hbm, psrc_hbm, pdst_hbm, out_hbm,
                   ibuf, rows, agg, gsem, isem):
        c = lax.axis_index("c")
        s = lax.axis_index("s")
        wid = c * 16 + s
        base = wid * BPW * LB

        def fire_idx(b, slot):
            off = base + b * LB

            @pl.when(off < E)
            def _():
                pltpu.async_copy(edge_hbm.at[0, pl.ds(off, LB)],
                                 ibuf.at[slot, 0, 0], isem)
                pltpu.async_copy(edge_hbm.at[1, pl.ds(off, LB)],
                                 ibuf.at[slot, 0, 1], isem)

            @pl.when(off >= E)
            def _():
                pltpu.async_copy(psrc_hbm.at[pl.ds(off - E, LB)],
                                 ibuf.at[slot, 0, 0], isem)
                pltpu.async_copy(pdst_hbm.at[pl.ds(off - E, LB)],
                                 ibuf.at[slot, 0, 1], isem)

        def wait_idx(slot):
            pltpu.make_async_copy(
                psrc_hbm.at[pl.ds(0, LB)], ibuf.at[slot, 0, 0], isem).wait()
            pltpu.make_async_copy(
                pdst_hbm.at[pl.ds(0, LB)], ibuf.at[slot, 0, 1], isem).wait()

        def fire_gather(slot):
            return pltpu.async_copy(
                x_hbm.at[ibuf.at[slot, 0, 0]], rows.at[slot], gsem)

        def wait_gather(slot):
            pltpu.make_async_copy(
                x_hbm.at[ibuf.at[slot, 0, 0]], rows.at[slot], gsem).wait()

        fire_idx(0, 0)
        fire_idx(1, 1)
        fire_idx(2, 2)

        def zrow(r, carry):
            for k in (0, 1, 2):
                for q in range(D // 16):
                    rows[k, r, pl.ds(q * 16, 16)] = jnp.zeros((16,), jnp.float32)
            return carry

        lax.fori_loop(0, LB, zrow, 0)
        zcps = [
            pltpu.async_copy(rows.at[k % 3],
                             agg.at[pl.ds(s * RPT + k * LB, LB)], gsem)
            for k in range(4)
        ]
        zcps.append(pltpu.async_copy(
            rows.at[1].at[pl.ds(0, RPT - 4 * LB)],
            agg.at[pl.ds(s * RPT + 4 * LB, RPT - 4 * LB)], gsem))
        for cp in zcps:
            cp.wait()

        wait_idx(0)
        fire_gather(0)
        wait_idx(1)
        fire_gather(1)
        plsc.subcore_barrier()

        def body(i, carry):
            for j in (0, 1, 2):
                b = i * 3 + j
                wait_idx((b + 2) % 3)
                fire_gather((b + 2) % 3)
                wait_gather(b % 3)
                pltpu.sync_copy(rows.at[b % 3],
                                agg.at[ibuf.at[b % 3, 0, 1]], add=True)
                fire_idx(b + 3, b % 3)
            return carry

        lax.fori_loop(0, BPW // 3, body, 0)

        wait_gather(0)
        wait_gather(1)
        wait_idx(2)
        plsc.subcore_barrier()

        pltpu.sync_copy(
            agg.at[pl.ds(s * RPT, RPT)],
            out_hbm.at[pl.ds(c * NP + s * RPT, RPT)],
        )

    return seg_kernel(x, edge_index, pad_src, pad_dst)


def _xw_body(x_ref, w1_ref, b1_ref, o_ref):
    o_ref[...] = jnp.dot(x_ref[...], w1_ref[...],
                         preferred_element_type=jnp.float32) + b1_ref[...]


def _mlp_body(xw_ref, p_ref, w1_ref, g1_ref, be1_ref,
              w2_ref, b2_ref, g2_ref, be2_ref, o_ref):
    agg = p_ref[0:N, :] + p_ref[NP:NP + N, :]
    h1 = xw_ref[...] + jnp.dot(agg, w1_ref[...],
                               preferred_element_type=jnp.float32)
    h1 = jnp.maximum(h1, 0.0)
    m1 = jnp.mean(h1, axis=0, keepdims=True)
    v1 = jnp.mean((h1 - m1) * (h1 - m1), axis=0, keepdims=True)
    h1 = (h1 - m1) * lax.rsqrt(v1 + 1e-5) * g1_ref[...] + be1_ref[...]
    h2 = jnp.dot(h1, w2_ref[...], preferred_element_type=jnp.float32) + b2_ref[...]
    m2 = jnp.mean(h2, axis=0, keepdims=True)
    v2 = jnp.mean((h2 - m2) * (h2 - m2), axis=0, keepdims=True)
    h2 = (h2 - m2) * lax.rsqrt(v2 + 1e-5) * g2_ref[...] + be2_ref[...]
    o_ref[...] = jnp.where(h2 >= 0, h2, 0.01 * h2)


def kernel(x, edge_index, W1, b1, g1, be1, W2, b2, g2, be2):
    partials = _segment_sum_sc(
        x, edge_index, jnp.asarray(_PAD_SRC), jnp.asarray(_PAD_DST))

    xw = pl.pallas_call(
        _xw_body,
        out_shape=jax.ShapeDtypeStruct((N, 32), jnp.float32),
    )(x, W1, b1.reshape(1, -1))

    out = pl.pallas_call(
        _mlp_body,
        out_shape=jax.ShapeDtypeStruct((N, 64), jnp.float32),
    )(
        xw, partials, W1,
        g1.reshape(1, -1), be1.reshape(1, -1),
        W2,
        b2.reshape(1, -1), g2.reshape(1, -1), be2.reshape(1, -1),
    )
    return out


# speedup vs baseline: 1.0001x; 1.0001x over previous
import functools

import jax
import jax.numpy as jnp
import numpy as np
from jax import lax
from jax.experimental import pallas as pl
from jax.experimental.pallas import tpu as pltpu
from jax.experimental.pallas import tpu_sc as plsc

N = 10000
E = 320000
D = 128
NW = 32
LB = 128
BPW = 81
SLOP = 3
NE_P = (NW * BPW + SLOP) * LB
NP = 10112
RPT = NP // 16

_PAD_SRC = np.asarray((np.arange(NE_P - E) * 997) % N, dtype=np.int32)
_PAD_DST = np.asarray(N + np.arange(NE_P - E) % (NP - N), dtype=np.int32)


def _segment_sum_sc(x, edge_index, pad_src, pad_dst):
    mesh = plsc.VectorSubcoreMesh(core_axis_name="c", subcore_axis_name="s")

    @functools.partial(
        pl.kernel,
        mesh=mesh,
        out_type=jax.ShapeDtypeStruct((2 * NP, D), jnp.float32),
        scratch_types=[
            pltpu.VMEM((3, 1, 2, LB), jnp.int32),
            pltpu.VMEM((3, LB, D), jnp.float32),
            pltpu.VMEM_SHARED((NP, D), jnp.float32),
            pltpu.SemaphoreType.DMA,
            pltpu.SemaphoreType.DMA,
        ],
    )
    def seg_kernel(x_hbm, edge_hbm, psrc_hbm, pdst_hbm, out_hbm,
                   ibuf, rows, agg, gsem, isem):
        c = lax.axis_index("c")
        s = lax.axis_index("s")
        wid = c * 16 + s
        base = wid * BPW * LB

        def fire_idx(b, slot):
            off = base + b * LB

            @pl.when(off < E)
            def _():
                pltpu.async_copy(edge_hbm.at[0, pl.ds(off, LB)],
                                 ibuf.at[slot, 0, 0], isem)
                pltpu.async_copy(edge_hbm.at[1, pl.ds(off, LB)],
                                 ibuf.at[slot, 0, 1], isem)

            @pl.when(off >= E)
            def _():
                pltpu.async_copy(psrc_hbm.at[pl.ds(off - E, LB)],
                                 ibuf.at[slot, 0, 0], isem)
                pltpu.async_copy(pdst_hbm.at[pl.ds(off - E, LB)],
                                 ibuf.at[slot, 0, 1], isem)

        def wait_idx(slot):
            pltpu.make_async_copy(
                psrc_hbm.at[pl.ds(0, LB)], ibuf.at[slot, 0, 0], isem).wait()
            pltpu.make_async_copy(
                pdst_hbm.at[pl.ds(0, LB)], ibuf.at[slot, 0, 1], isem).wait()

        def fire_gather(slot):
            return pltpu.async_copy(
                x_hbm.at[ibuf.at[slot, 0, 0]], rows.at[slot], gsem)

        def wait_gather(slot):
            pltpu.make_async_copy(
                x_hbm.at[ibuf.at[slot, 0, 0]], rows.at[slot], gsem).wait()

        fire_idx(0, 0)
        fire_idx(1, 1)
        fire_idx(2, 2)

        def zrow(r, carry):
            for k in (0, 1, 2):
                for q in range(D // 16):
                    rows[k, r, pl.ds(q * 16, 16)] = jnp.zeros((16,), jnp.float32)
            return carry

        lax.fori_loop(0, LB, zrow, 0)
        zcps = [
            pltpu.async_copy(rows.at[k % 3],
                             agg.at[pl.ds(s * RPT + k * LB, LB)], gsem)
            for k in range(4)
        ]
        zcps.append(pltpu.async_copy(
            rows.at[1].at[pl.ds(0, RPT - 4 * LB)],
            agg.at[pl.ds(s * RPT + 4 * LB, RPT - 4 * LB)], gsem))
        for cp in zcps:
            cp.wait()

        wait_idx(0)
        fire_gather(0)
        wait_idx(1)
        fire_gather(1)
        plsc.subcore_barrier()

        def body(i, carry):
            for j in (0, 1, 2):
                b = i * 3 + j
                wait_idx((b + 2) % 3)
                fire_gather((b + 2) % 3)
                wait_gather(b % 3)
                pltpu.sync_copy(rows.at[b % 3],
                                agg.at[ibuf.at[b % 3, 0, 1]], add=True)
                fire_idx(b + 3, b % 3)
            return carry

        lax.fori_loop(0, BPW // 3, body, 0)

        wait_gather(0)
        wait_gather(1)
        wait_idx(2)
        plsc.subcore_barrier()

        pltpu.sync_copy(
            agg.at[pl.ds(s * RPT, RPT)],
            out_hbm.at[pl.ds(c * NP + s * RPT, RPT)],
        )

    return seg_kernel(x, edge_index, pad_src, pad_dst)


def _mlp_body(x_ref, p_ref, w1_ref, b1_ref, g1_ref, be1_ref,
              w2_ref, b2_ref, g2_ref, be2_ref, o_ref):
    h = x_ref[...] + p_ref[0:N, :] + p_ref[NP:NP + N, :]
    h1 = jnp.dot(h, w1_ref[...], preferred_element_type=jnp.float32) + b1_ref[...]
    h1 = jnp.maximum(h1, 0.0)
    m1 = jnp.mean(h1, axis=0, keepdims=True)
    v1 = jnp.mean((h1 - m1) * (h1 - m1), axis=0, keepdims=True)
    h1 = (h1 - m1) * lax.rsqrt(v1 + 1e-5) * g1_ref[...] + be1_ref[...]
    h2 = jnp.dot(h1, w2_ref[...], preferred_element_type=jnp.float32) + b2_ref[...]
    m2 = jnp.mean(h2, axis=0, keepdims=True)
    v2 = jnp.mean((h2 - m2) * (h2 - m2), axis=0, keepdims=True)
    h2 = (h2 - m2) * lax.rsqrt(v2 + 1e-5) * g2_ref[...] + be2_ref[...]
    o_ref[...] = jnp.where(h2 >= 0, h2, 0.01 * h2)


def kernel(x, edge_index, W1, b1, g1, be1, W2, b2, g2, be2):
    partials = _segment_sum_sc(
        x, edge_index, jnp.asarray(_PAD_SRC), jnp.asarray(_PAD_DST))

    out = pl.pallas_call(
        _mlp_body,
        out_shape=jax.ShapeDtypeStruct((N, 64), jnp.float32),
    )(
        x, partials, W1,
        b1.reshape(1, -1), g1.reshape(1, -1), be1.reshape(1, -1),
        W2,
        b2.reshape(1, -1), g2.reshape(1, -1), be2.reshape(1, -1),
    )
    return out
